# baseline (speedup 1.0000x reference)
import functools

import jax
import jax.numpy as jnp
from jax import lax
from jax.experimental import pallas as pl
from jax.experimental.pallas import tpu as pltpu
from jax.experimental.pallas import tpu_sc as plsc


def _sc_gather(emb_table, flat_idx, n_rows, d):
    try:
        info = plsc.get_sparse_core_info()
        nc, ns = info.num_cores, info.num_subcores
    except Exception:
        nc, ns = 2, 16
    nw = nc * ns
    assert n_rows % (8 * nw) == 0
    b_per_w = n_rows // nw
    mesh = plsc.VectorSubcoreMesh(core_axis_name="c", subcore_axis_name="s")

    @functools.partial(
        pl.kernel,
        mesh=mesh,
        compiler_params=pltpu.CompilerParams(use_tc_tiling_on_sc=False),
        out_type=jax.ShapeDtypeStruct((n_rows, d), jnp.float32),
        scratch_types=[
            pltpu.VMEM((b_per_w,), jnp.int32),
            pltpu.VMEM((b_per_w, d), jnp.float32),
            pltpu.SemaphoreType.DMA,
        ],
    )
    def gather_kernel(table_hbm, idx_hbm, out_hbm, idx_v, rows_v, sem):
        wid = lax.axis_index("s") * nc + lax.axis_index("c")
        base = wid * b_per_w
        pltpu.sync_copy(idx_hbm.at[pl.ds(base, b_per_w)], idx_v)
        pltpu.async_copy(table_hbm.at[idx_v], rows_v, sem).wait()
        pltpu.sync_copy(rows_v, out_hbm.at[pl.ds(base, b_per_w)])

    return gather_kernel(emb_table, flat_idx)


def _pool_matmul_body(g_ref, w_ref, o_ref, pooled_ref, *, ctx):
    @pl.when(pl.program_id(0) == 0)
    def _():
        pooled_ref[...] = jnp.sum(g_ref[...], axis=0) * (1.0 / ctx)

    o_ref[...] = lax.dot_general(
        pooled_ref[...],
        w_ref[...],
        dimension_numbers=(((1,), (1,)), ((), ())),
        preferred_element_type=jnp.float32,
        precision=lax.Precision.HIGHEST,
    )


def _pool_matmul(gathered3, w_out, row_block):
    ctx, batch, d = gathered3.shape
    vocab = w_out.shape[0]
    grid = pl.cdiv(vocab, row_block)
    return pl.pallas_call(
        functools.partial(_pool_matmul_body, ctx=ctx),
        grid=(grid,),
        in_specs=[
            pl.BlockSpec((ctx, batch, d), lambda i: (0, 0, 0)),
            pl.BlockSpec((row_block, d), lambda i: (i, 0)),
        ],
        out_specs=pl.BlockSpec((batch, row_block), lambda i: (0, i)),
        out_shape=jax.ShapeDtypeStruct((batch, vocab), jnp.float32),
        scratch_shapes=[pltpu.VMEM((batch, d), jnp.float32)],
    )(gathered3, w_out)


def kernel(x, emb_table, W_out):
    batch, ctx = x.shape
    vocab, d = W_out.shape
    flat_idx = x.astype(jnp.int32).T.reshape(-1)
    gathered = _sc_gather(emb_table, flat_idx, batch * ctx, d)
    gathered3 = gathered.reshape(ctx, batch, d)
    return _pool_matmul(gathered3, W_out, row_block=2048)


# speedup vs baseline: 1.3020x; 1.3020x over previous
import functools

import jax
import jax.numpy as jnp
from jax import lax
from jax.experimental import pallas as pl
from jax.experimental.pallas import tpu as pltpu
from jax.experimental.pallas import tpu_sc as plsc


def _sc_gather(emb_table, flat_idx, n_rows, d):
    try:
        info = plsc.get_sparse_core_info()
        nc, ns = info.num_cores, info.num_subcores
    except Exception:
        nc, ns = 2, 16
    nw = nc * ns
    assert n_rows % (8 * nw) == 0
    b_per_w = n_rows // nw
    mesh = plsc.VectorSubcoreMesh(core_axis_name="c", subcore_axis_name="s")

    @functools.partial(
        pl.kernel,
        mesh=mesh,
        compiler_params=pltpu.CompilerParams(use_tc_tiling_on_sc=False),
        out_type=jax.ShapeDtypeStruct((n_rows, d), jnp.float32),
        scratch_types=[
            pltpu.VMEM((b_per_w,), jnp.int32),
            pltpu.VMEM((b_per_w, d), jnp.float32),
            pltpu.SemaphoreType.DMA,
        ],
    )
    def gather_kernel(table_hbm, idx_hbm, out_hbm, idx_v, rows_v, sem):
        wid = lax.axis_index("s") * nc + lax.axis_index("c")
        base = wid * b_per_w
        pltpu.sync_copy(idx_hbm.at[pl.ds(base, b_per_w)], idx_v)
        pltpu.async_copy(table_hbm.at[idx_v], rows_v, sem).wait()
        pltpu.sync_copy(rows_v, out_hbm.at[pl.ds(base, b_per_w)])

    return gather_kernel(emb_table, flat_idx)


def _pool_matmul_body(g_ref, w_ref, o_ref, pooled_ref, *, ctx):
    @pl.when(pl.program_id(0) == 0)
    def _():
        pooled_ref[...] = jnp.sum(g_ref[...], axis=0) * (1.0 / ctx)

    o_ref[...] = lax.dot_general(
        pooled_ref[...],
        w_ref[...],
        dimension_numbers=(((1,), (1,)), ((), ())),
        preferred_element_type=jnp.float32,
        precision=lax.Precision.DEFAULT,
    )


def _pool_matmul(gathered3, w_out, row_block):
    ctx, batch, d = gathered3.shape
    vocab = w_out.shape[0]
    grid = pl.cdiv(vocab, row_block)
    return pl.pallas_call(
        functools.partial(_pool_matmul_body, ctx=ctx),
        grid=(grid,),
        in_specs=[
            pl.BlockSpec((ctx, batch, d), lambda i: (0, 0, 0)),
            pl.BlockSpec((row_block, d), lambda i: (i, 0)),
        ],
        out_specs=pl.BlockSpec((batch, row_block), lambda i: (0, i)),
        out_shape=jax.ShapeDtypeStruct((batch, vocab), jnp.float32),
        scratch_shapes=[pltpu.VMEM((batch, d), jnp.float32)],
    )(gathered3, w_out)


def kernel(x, emb_table, W_out):
    batch, ctx = x.shape
    vocab, d = W_out.shape
    flat_idx = x.astype(jnp.int32).T.reshape(-1)
    gathered = _sc_gather(emb_table, flat_idx, batch * ctx, d)
    gathered3 = gathered.reshape(ctx, batch, d)
    return _pool_matmul(gathered3, W_out, row_block=2048)
